# baseline (speedup 1.0000x reference)
import jax
import jax.numpy as jnp
from jax.experimental import pallas as pl
from jax.experimental.pallas import tpu as pltpu


_N = 10000
_E = 160000
_NPAD = 10240


def _mm_bias_body(x_ref, w_ref, b_ref, o_ref):
    o_ref[...] = (
        jnp.dot(x_ref[...], w_ref[...], preferred_element_type=jnp.float32)
        + b_ref[...]
    )


def _mm_bias(xp, wcat, bcat):
    return pl.pallas_call(
        _mm_bias_body,
        grid=(20,),
        in_specs=[
            pl.BlockSpec((512, 256), lambda i: (i, 0)),
            pl.BlockSpec((256, 1024), lambda i: (0, 0)),
            pl.BlockSpec((1, 1024), lambda i: (0, 0)),
        ],
        out_specs=pl.BlockSpec((512, 1024), lambda i: (i, 0)),
        out_shape=jax.ShapeDtypeStruct((_NPAD, 1024), jnp.float32),
    )(xp, wcat, bcat)


def _rowdot_scale_body(a_ref, b_ref, o_ref):
    o_ref[...] = jnp.sum(a_ref[...] * b_ref[...], axis=-1) * (1.0 / 16.0)


def _rowdot_sigmoid_body(a_ref, b_ref, o_ref):
    o_ref[...] = jax.nn.sigmoid(jnp.sum(a_ref[...] * b_ref[...], axis=-1))


_EPAD = 163840


def _rowdot(a, b, body, d):
    a3 = a.reshape(1280, 128, d)
    b3 = b.reshape(1280, 128, d)
    out = pl.pallas_call(
        body,
        grid=(80,),
        in_specs=[
            pl.BlockSpec((16, 128, d), lambda i: (i, 0, 0)),
            pl.BlockSpec((16, 128, d), lambda i: (i, 0, 0)),
        ],
        out_specs=pl.BlockSpec((16, 128), lambda i: (i, 0)),
        out_shape=jax.ShapeDtypeStruct((1280, 128), jnp.float32),
    )(a3, b3)
    return out.reshape(_EPAD)[: _E]


def _head_body(z_ref, w1_ref, b1_ref, w2_ref, b2_ref, o_ref, acc_ref):
    i = pl.program_id(0)

    @pl.when(i == 0)
    def _():
        acc_ref[...] = jnp.zeros_like(acc_ref)

    y1 = jnp.max(z_ref[...], axis=1)
    acc_ref[...] += jnp.dot(
        y1[None, :], w1_ref[...], preferred_element_type=jnp.float32
    )

    @pl.when(i == 9)
    def _():
        y2 = jax.nn.relu(acc_ref[...] + b1_ref[...])
        o_ref[...] = (
            jnp.dot(y2, w2_ref[...], preferred_element_type=jnp.float32)
            + b2_ref[...]
        )


def _head(z, w1, b1, w2, b2):
    return pl.pallas_call(
        _head_body,
        grid=(10,),
        in_specs=[
            pl.BlockSpec((1000, 256), lambda i: (i, 0)),
            pl.BlockSpec((1000, 256), lambda i: (i, 0)),
            pl.BlockSpec((1, 256), lambda i: (0, 0)),
            pl.BlockSpec((256, 64), lambda i: (0, 0)),
            pl.BlockSpec((1, 64), lambda i: (0, 0)),
        ],
        out_specs=pl.BlockSpec((1, 64), lambda i: (0, 0)),
        out_shape=jax.ShapeDtypeStruct((1, 64), jnp.float32),
        scratch_shapes=[pltpu.VMEM((1, 256), jnp.float32)],
    )(z, w1, b1[None, :], w2, b2[None, :])


def _pad_rows(x):
    return jnp.pad(x, ((0, _NPAD - _N), (0, 0)))


def _pad_idx(i):
    return jnp.pad(i, (0, _EPAD - _E))


def _tconv_layer(xp, src, dst, srcp, dstp, wcat, bcat):
    qkvs = _mm_bias(xp, wcat, bcat)[: _N]
    q = qkvs[:, 0:256]
    k = qkvs[:, 256:512]
    v = qkvs[:, 512:768]
    s = qkvs[:, 768:1024]
    alpha = _rowdot(q[dstp], k[srcp], _rowdot_scale_body, 256)
    amax = jax.ops.segment_max(alpha, dst, num_segments=_N)
    amax = jnp.where(jnp.isfinite(amax), amax, 0.0)
    ex = jnp.exp(alpha - amax[dst])
    den = jax.ops.segment_sum(ex, dst, num_segments=_N)
    w = ex / (den[dst] + 1e-16)
    agg = jax.ops.segment_sum(v[src] * w[:, None], dst, num_segments=_N)
    return agg + s


def kernel(x, train_edge_index_0, train_edge_index_1, Wq1, bq1, Wk1, bk1, Wv1, bv1, Ws1, bs1, Wq3, bq3, Wk3, bk3, Wv3, bv3, Ws3, bs3, W1, b1, W2, b2):
    ei0 = train_edge_index_0.astype(jnp.int32)
    ei1 = train_edge_index_1.astype(jnp.int32)
    src = ei1[0]
    dst = ei1[1]
    srcp = _pad_idx(src)
    dstp = _pad_idx(dst)

    wcat1 = jnp.concatenate([Wq1, Wk1, Wv1, Ws1], axis=1)
    bcat1 = jnp.concatenate([bq1, bk1, bv1, bs1])[None, :]
    wcat3 = jnp.concatenate([Wq3, Wk3, Wv3, Ws3], axis=1)
    bcat3 = jnp.concatenate([bq3, bk3, bv3, bs3])[None, :]

    z1 = jax.nn.relu(_tconv_layer(_pad_rows(x), src, dst, srcp, dstp, wcat1, bcat1))
    z = _tconv_layer(_pad_rows(z1), src, dst, srcp, dstp, wcat3, bcat3)

    y = _head(z, W1, b1, W2, b2)

    adj0 = _rowdot(z[_pad_idx(ei0[0])], z[_pad_idx(ei0[1])], _rowdot_sigmoid_body, 256)
    adj1 = _rowdot(z[srcp], z[dstp], _rowdot_sigmoid_body, 256)
    return (adj0, adj1, z, y)


# speedup vs baseline: 1.0043x; 1.0043x over previous
import jax
import jax.numpy as jnp
from jax.experimental import pallas as pl
from jax.experimental.pallas import tpu as pltpu


_N = 10000
_E = 160000
_NPAD = 10240


def _mm_bias_body(x_ref, w_ref, b_ref, q_ref, k_ref, v_ref, s_ref):
    r = (
        jnp.dot(x_ref[...], w_ref[...], preferred_element_type=jnp.float32)
        + b_ref[...]
    )
    q_ref[...] = r[:, 0:256]
    k_ref[...] = r[:, 256:512]
    v_ref[...] = r[:, 512:768]
    s_ref[...] = r[:, 768:1024]


def _mm_bias(x, wcat, bcat):
    ospec = pl.BlockSpec((400, 256), lambda i: (i, 0))
    oshape = jax.ShapeDtypeStruct((_N, 256), jnp.float32)
    return pl.pallas_call(
        _mm_bias_body,
        grid=(25,),
        in_specs=[
            pl.BlockSpec((400, 256), lambda i: (i, 0)),
            pl.BlockSpec((256, 1024), lambda i: (0, 0)),
            pl.BlockSpec((1, 1024), lambda i: (0, 0)),
        ],
        out_specs=[ospec, ospec, ospec, ospec],
        out_shape=[oshape, oshape, oshape, oshape],
    )(x, wcat, bcat)


def _rowdot_scale_body(a_ref, b_ref, o_ref):
    o_ref[...] = jnp.sum(a_ref[...] * b_ref[...], axis=-1) * (1.0 / 16.0)


def _rowdot_sigmoid_body(a_ref, b_ref, o_ref):
    o_ref[...] = jax.nn.sigmoid(jnp.sum(a_ref[...] * b_ref[...], axis=-1))


_EPAD = 163840


def _rowdot(a, b, body, d):
    a3 = a.reshape(1280, 128, d)
    b3 = b.reshape(1280, 128, d)
    out = pl.pallas_call(
        body,
        grid=(80,),
        in_specs=[
            pl.BlockSpec((16, 128, d), lambda i: (i, 0, 0)),
            pl.BlockSpec((16, 128, d), lambda i: (i, 0, 0)),
        ],
        out_specs=pl.BlockSpec((16, 128), lambda i: (i, 0)),
        out_shape=jax.ShapeDtypeStruct((1280, 128), jnp.float32),
    )(a3, b3)
    return out.reshape(_EPAD)[: _E]


def _head_body(z_ref, w1_ref, b1_ref, w2_ref, b2_ref, o_ref, acc_ref):
    i = pl.program_id(0)

    @pl.when(i == 0)
    def _():
        acc_ref[...] = jnp.zeros_like(acc_ref)

    y1 = jnp.max(z_ref[...], axis=1)
    acc_ref[...] += jnp.dot(
        y1[None, :], w1_ref[...], preferred_element_type=jnp.float32
    )

    @pl.when(i == 9)
    def _():
        y2 = jax.nn.relu(acc_ref[...] + b1_ref[...])
        o_ref[...] = (
            jnp.dot(y2, w2_ref[...], preferred_element_type=jnp.float32)
            + b2_ref[...]
        )


def _head(z, w1, b1, w2, b2):
    return pl.pallas_call(
        _head_body,
        grid=(10,),
        in_specs=[
            pl.BlockSpec((1000, 256), lambda i: (i, 0)),
            pl.BlockSpec((1000, 256), lambda i: (i, 0)),
            pl.BlockSpec((1, 256), lambda i: (0, 0)),
            pl.BlockSpec((256, 64), lambda i: (0, 0)),
            pl.BlockSpec((1, 64), lambda i: (0, 0)),
        ],
        out_specs=pl.BlockSpec((1, 64), lambda i: (0, 0)),
        out_shape=jax.ShapeDtypeStruct((1, 64), jnp.float32),
        scratch_shapes=[pltpu.VMEM((1, 256), jnp.float32)],
    )(z, w1, b1[None, :], w2, b2[None, :])


def _pad_idx(i):
    return jnp.pad(i, (0, _EPAD - _E))


def _tconv_layer(x, src, dst, srcp, dstp, wcat, bcat):
    q, k, v, s = _mm_bias(x, wcat, bcat)
    alpha = _rowdot(q[dstp], k[srcp], _rowdot_scale_body, 256)
    amax = jax.ops.segment_max(alpha, dst, num_segments=_N)
    amax = jnp.where(jnp.isfinite(amax), amax, 0.0)
    ex = jnp.exp(alpha - amax[dst])
    den = jax.ops.segment_sum(ex, dst, num_segments=_N)
    w = ex / (den[dst] + 1e-16)
    agg = jax.ops.segment_sum(v[src] * w[:, None], dst, num_segments=_N)
    return agg + s


def kernel(x, train_edge_index_0, train_edge_index_1, Wq1, bq1, Wk1, bk1, Wv1, bv1, Ws1, bs1, Wq3, bq3, Wk3, bk3, Wv3, bv3, Ws3, bs3, W1, b1, W2, b2):
    ei0 = train_edge_index_0.astype(jnp.int32)
    ei1 = train_edge_index_1.astype(jnp.int32)
    src = ei1[0]
    dst = ei1[1]
    srcp = _pad_idx(src)
    dstp = _pad_idx(dst)

    wcat1 = jnp.concatenate([Wq1, Wk1, Wv1, Ws1], axis=1)
    bcat1 = jnp.concatenate([bq1, bk1, bv1, bs1])[None, :]
    wcat3 = jnp.concatenate([Wq3, Wk3, Wv3, Ws3], axis=1)
    bcat3 = jnp.concatenate([bq3, bk3, bv3, bs3])[None, :]

    z1 = jax.nn.relu(_tconv_layer(x, src, dst, srcp, dstp, wcat1, bcat1))
    z = _tconv_layer(z1, src, dst, srcp, dstp, wcat3, bcat3)

    y = _head(z, W1, b1, W2, b2)

    adj0 = _rowdot(z[_pad_idx(ei0[0])], z[_pad_idx(ei0[1])], _rowdot_sigmoid_body, 256)
    adj1 = _rowdot(z[srcp], z[dstp], _rowdot_sigmoid_body, 256)
    return (adj0, adj1, z, y)
